# baseline (speedup 1.0000x reference)
import jax
import jax.numpy as jnp
from jax.experimental import pallas as pl
from jax.experimental.pallas import tpu as pltpu

B = 8
N = 16384
C = 3
NPTS = 1024
BIG = 1e10


def _fps_body(x_ref, m_ref, out_ref, dists_ref):
    iota_n = jax.lax.broadcasted_iota(jnp.int32, (B, N), 1)
    iota_p = jax.lax.broadcasted_iota(jnp.int32, (B, NPTS), 1)
    col = jax.lax.broadcasted_iota(jnp.int32, (B, 24), 1)
    row3 = 3 * jax.lax.broadcasted_iota(jnp.int32, (B, 24), 0)

    qx0 = x_ref[0][:, 0:1]
    qy0 = x_ref[1][:, 0:1]
    qz0 = x_ref[2][:, 0:1]
    zeros_p = jnp.zeros((B, NPTS), dtype=jnp.float32)
    out_ref[0] = jnp.where(iota_p == 0, qx0, zeros_p)
    out_ref[1] = jnp.where(iota_p == 0, qy0, zeros_p)
    out_ref[2] = jnp.where(iota_p == 0, qz0, zeros_p)
    dists_ref[...] = jnp.full((B, N), BIG, dtype=jnp.float32)

    def body(i, q):
        qx, qy, qz = q
        dx = x_ref[0] - qx
        dy = x_ref[1] - qy
        dz = x_ref[2] - qz
        d = dx * dx + dy * dy + dz * dz
        dists = jnp.minimum(dists_ref[...], d)
        dists_ref[...] = dists
        m = jnp.max(dists, axis=1, keepdims=True)
        nxt = jnp.min(jnp.where(dists == m, iota_n, N), axis=1, keepdims=True)
        onehot = jnp.where(iota_n == nxt, 1.0, 0.0)
        picked = jax.lax.dot_general(
            onehot, m_ref[...],
            (((1,), (0,)), ((), ())),
            preferred_element_type=jnp.float32,
        )
        qx = jnp.sum(jnp.where(col == row3, picked, 0.0), axis=1, keepdims=True)
        qy = jnp.sum(jnp.where(col == row3 + 1, picked, 0.0), axis=1, keepdims=True)
        qz = jnp.sum(jnp.where(col == row3 + 2, picked, 0.0), axis=1, keepdims=True)
        osel = iota_p == (i + 1)
        out_ref[0] = jnp.where(osel, qx, out_ref[0])
        out_ref[1] = jnp.where(osel, qy, out_ref[1])
        out_ref[2] = jnp.where(osel, qz, out_ref[2])
        return qx, qy, qz

    jax.lax.fori_loop(0, NPTS - 1, body, (qx0, qy0, qz0))


def kernel(x):
    xt = jnp.transpose(x, (1, 0, 2))
    m = jnp.transpose(x, (2, 0, 1)).reshape(N, B * C)
    out = pl.pallas_call(
        _fps_body,
        out_shape=jax.ShapeDtypeStruct((C, B, NPTS), jnp.float32),
        scratch_shapes=[pltpu.VMEM((B, N), jnp.float32)],
    )(xt, m)
    return jnp.transpose(out, (1, 0, 2))


# speedup vs baseline: 1.4304x; 1.4304x over previous
import jax
import jax.numpy as jnp
from jax.experimental import pallas as pl
from jax.experimental.pallas import tpu as pltpu

B = 8
N = 16384
C = 3
NPTS = 1024
BIG = 1e10


def _fps_body(x_ref, out_ref, dists_ref):
    iota_n = jax.lax.broadcasted_iota(jnp.int32, (B, N), 1)
    iota_p = jax.lax.broadcasted_iota(jnp.int32, (B, NPTS), 1)

    qx0 = x_ref[0][:, 0:1]
    qy0 = x_ref[1][:, 0:1]
    qz0 = x_ref[2][:, 0:1]
    zeros_p = jnp.zeros((B, NPTS), dtype=jnp.float32)
    out_ref[0] = jnp.where(iota_p == 0, qx0, zeros_p)
    out_ref[1] = jnp.where(iota_p == 0, qy0, zeros_p)
    out_ref[2] = jnp.where(iota_p == 0, qz0, zeros_p)
    dists_ref[...] = jnp.full((B, N), BIG, dtype=jnp.float32)

    def body(i, q):
        qx, qy, qz = q
        dx = x_ref[0] - qx
        dy = x_ref[1] - qy
        dz = x_ref[2] - qz
        d = dx * dx + dy * dy + dz * dz
        dists = jnp.minimum(dists_ref[...], d)
        dists_ref[...] = dists
        m = jnp.max(dists, axis=1, keepdims=True)
        nxt = jnp.min(jnp.where(dists == m, iota_n, N), axis=1, keepdims=True)
        sel = iota_n == nxt
        qx = jnp.max(jnp.where(sel, x_ref[0], -BIG), axis=1, keepdims=True)
        qy = jnp.max(jnp.where(sel, x_ref[1], -BIG), axis=1, keepdims=True)
        qz = jnp.max(jnp.where(sel, x_ref[2], -BIG), axis=1, keepdims=True)
        osel = iota_p == (i + 1)
        out_ref[0] = jnp.where(osel, qx, out_ref[0])
        out_ref[1] = jnp.where(osel, qy, out_ref[1])
        out_ref[2] = jnp.where(osel, qz, out_ref[2])
        return qx, qy, qz

    jax.lax.fori_loop(0, NPTS - 1, body, (qx0, qy0, qz0))


def kernel(x):
    xt = jnp.transpose(x, (1, 0, 2))
    out = pl.pallas_call(
        _fps_body,
        out_shape=jax.ShapeDtypeStruct((C, B, NPTS), jnp.float32),
        scratch_shapes=[pltpu.VMEM((B, N), jnp.float32)],
    )(xt)
    return jnp.transpose(out, (1, 0, 2))
